# Initial kernel scaffold; baseline (speedup 1.0000x reference)
import functools

import jax
import jax.numpy as jnp
from jax import lax
from jax.experimental import pallas as pl
from jax.experimental.pallas import tpu as pltpu
from jax.experimental.pallas import tpu_sc as plsc

_B, _L, _D = 16384, 200, 32
_N = _B * _L
_NC, _NS = 2, 16
_NW = _NC * _NS
_PER_W = _N // _NW
_ROWS = 128
_KSUB = 8
_CHUNK = _KSUB * _ROWS
_NCHUNK = _PER_W // _CHUNK
_ROW_STRIDE = _PER_W // _ROWS


def _emb_body(ys_hbm, table_hbm, out_hbm, idx_v, rows_v, gsem):
    wid = lax.axis_index("s") * _NC + lax.axis_index("c")
    row_base0 = wid * _ROW_STRIDE

    def body(g, carry):
        row_base = row_base0 + g * _KSUB
        pltpu.sync_copy(ys_hbm.at[pl.ds(row_base, _KSUB)], idx_v)
        copies = [
            pltpu.async_copy(table_hbm.at[idx_v.at[j]], rows_v.at[j], gsem)
            for j in range(_KSUB)
        ]
        for c in copies:
            c.wait()
        pltpu.sync_copy(rows_v, out_hbm.at[pl.ds(row_base, _KSUB)])
        return carry

    lax.fori_loop(0, _NCHUNK, body, 0)


@jax.jit
def _embed(ys2, table):
    mesh = plsc.VectorSubcoreMesh(core_axis_name="c", subcore_axis_name="s")
    f = pl.kernel(
        _emb_body,
        out_type=jax.ShapeDtypeStruct((_N // _ROWS, _ROWS, _D), jnp.float32),
        mesh=mesh,
        scratch_types=[
            pltpu.VMEM((_KSUB, _ROWS), jnp.int32),
            pltpu.VMEM((_KSUB, _ROWS, _D), jnp.float32),
            pltpu.SemaphoreType.DMA,
        ],
    )
    return f(ys2, table)


def kernel(ys, table):
    ys2 = ys.astype(jnp.int32).reshape(_N // _ROWS, _ROWS)
    out = _embed(ys2, table)
    return out.reshape(_B, _L, _D)


# speedup vs baseline: 4.8022x; 4.8022x over previous
import functools

import jax
import jax.numpy as jnp
from jax import lax
from jax.experimental import pallas as pl
from jax.experimental.pallas import tpu as pltpu
from jax.experimental.pallas import tpu_sc as plsc

_B, _L, _D = 16384, 200, 32
_N = _B * _L
_NC, _NS = 2, 16
_NW = _NC * _NS
_PER_W = _N // _NW
_ROWS = 128
_KSUB = 8
_CHUNK = _KSUB * _ROWS
_NCHUNK = _PER_W // _CHUNK
_ROW_STRIDE = _PER_W // _ROWS


def _emb_body(ys_hbm, table_hbm, out_hbm, idx_v, rows_v, gsem):
    wid = lax.axis_index("s") * _NC + lax.axis_index("c")
    row_base0 = wid * _ROW_STRIDE

    def body(g, carry):
        row_base = row_base0 + g * _KSUB
        pltpu.sync_copy(ys_hbm.at[pl.ds(row_base, _KSUB)], idx_v)
        copies = [
            pltpu.async_copy(table_hbm.at[idx_v.at[j]], rows_v.at[j], gsem)
            for j in range(_KSUB)
        ]
        for c in copies:
            c.wait()
        pltpu.sync_copy(rows_v, out_hbm.at[pl.ds(row_base, _KSUB)])
        return carry

    lax.fori_loop(0, _NCHUNK, body, 0)


@jax.jit
def _embed(ys2, table):
    mesh = plsc.VectorSubcoreMesh(core_axis_name="c", subcore_axis_name="s")
    f = pl.kernel(
        _emb_body,
        out_type=jax.ShapeDtypeStruct((_N // _ROWS, _ROWS, _D), jnp.float32),
        mesh=mesh,
        scratch_types=[
            pltpu.VMEM((_KSUB, _ROWS), jnp.int32),
            pltpu.VMEM((_KSUB, _ROWS, _D), jnp.float32),
            pltpu.SemaphoreType.DMA,
        ],
        compiler_params=pltpu.CompilerParams(use_tc_tiling_on_sc=False),
    )
    return f(ys2, table)


def kernel(ys, table):
    ys2 = ys.astype(jnp.int32).reshape(_N // _ROWS, _ROWS)
    out = _embed(ys2, table)
    return out.reshape(_B, _L, _D)


# speedup vs baseline: 4.9442x; 1.0296x over previous
import jax
import jax.numpy as jnp
from jax import lax
from jax.experimental import pallas as pl
from jax.experimental.pallas import tpu as pltpu
from jax.experimental.pallas import tpu_sc as plsc

_B, _L, _D = 16384, 200, 32
_N = _B * _L
_NC, _NS = 2, 16
_NW = _NC * _NS
_PER_W = _N // _NW
_ROWS = 128
_KSUB = 8
_CHUNK = _KSUB * _ROWS
_NCHUNK = _PER_W // _CHUNK
_ROW_STRIDE = _PER_W // _ROWS
_NBUF = 2
_NGROUPS = _NCHUNK // _NBUF


def _emb_body(ys_hbm, table_hbm, out_hbm,
              idx0, idx1, rows0, rows1, gs0, gs1, os0, os1):
    idx = (idx0, idx1)
    rows = (rows0, rows1)
    gsem = (gs0, gs1)
    osem = (os0, os1)
    wid = lax.axis_index("s") * _NC + lax.axis_index("c")
    rb0 = wid * _ROW_STRIDE

    def fire(b, row_base):
        pltpu.sync_copy(ys_hbm.at[pl.ds(row_base, _KSUB)], idx[b])
        for j in range(_KSUB):
            pltpu.async_copy(table_hbm.at[idx[b].at[j]], rows[b].at[j], gsem[b])

    def wait_gather(b, row_base):
        pltpu.make_async_copy(
            out_hbm.at[pl.ds(row_base, _KSUB)], rows[b], gsem[b]).wait()

    for b in range(_NBUF):
        fire(b, rb0 + b * _KSUB)

    def body(i, carry):
        for b in range(_NBUF):
            g = i * _NBUF + b
            row_base = rb0 + g * _KSUB
            wait_gather(b, row_base)
            wcp = pltpu.make_async_copy(
                rows[b], out_hbm.at[pl.ds(row_base, _KSUB)], osem[b])
            wcp.start()
            wcp.wait()

            @pl.when(i < _NGROUPS - 1)
            def _():
                fire(b, row_base + _NBUF * _KSUB)

        return carry

    lax.fori_loop(0, _NGROUPS, body, 0)


@jax.jit
def _embed(ys2, table):
    mesh = plsc.VectorSubcoreMesh(core_axis_name="c", subcore_axis_name="s")
    f = pl.kernel(
        _emb_body,
        out_type=jax.ShapeDtypeStruct((_N // _ROWS, _ROWS, _D), jnp.float32),
        mesh=mesh,
        scratch_types=[
            pltpu.VMEM((_KSUB, _ROWS), jnp.int32),
            pltpu.VMEM((_KSUB, _ROWS), jnp.int32),
            pltpu.VMEM((_KSUB, _ROWS, _D), jnp.float32),
            pltpu.VMEM((_KSUB, _ROWS, _D), jnp.float32),
            pltpu.SemaphoreType.DMA,
            pltpu.SemaphoreType.DMA,
            pltpu.SemaphoreType.DMA,
            pltpu.SemaphoreType.DMA,
        ],
        compiler_params=pltpu.CompilerParams(use_tc_tiling_on_sc=False),
    )
    return f(ys2, table)


def kernel(ys, table):
    ys2 = ys.astype(jnp.int32).reshape(_N // _ROWS, _ROWS)
    out = _embed(ys2, table)
    return out.reshape(_B, _L, _D)
